# baseline (speedup 1.0000x reference)
import functools

import jax
import jax.numpy as jnp
from jax import lax
from jax.experimental import pallas as pl
from jax.experimental.pallas import tpu as pltpu
from jax.experimental.pallas import tpu_sc as plsc

N = 10000
NP = 10240
E = 320000
D = 128
D_OUT = 64

NC = 2
NS = 16
NW = NC * NS
CK = 128
CH = 80
EP = NW * CH * CK
DUMP = NP - 8
NB = 2
RPS = NP // NS
ZR = 128

R = 1000


def _mesh():
    return plsc.VectorSubcoreMesh(core_axis_name="c", subcore_axis_name="s")


def _sc_degrees(srcw, dstw, ones_h, zeros_h):
    @functools.partial(
        pl.kernel,
        mesh=_mesh(),
        out_type=jax.ShapeDtypeStruct((NC, 2, NP), jnp.float32),
        scratch_types=[
            pltpu.VMEM((CH, CK), jnp.int32),
            pltpu.VMEM((CH, CK), jnp.int32),
            pltpu.VMEM((CK,), jnp.float32),
            pltpu.VMEM_SHARED((NP,), jnp.float32),
            pltpu.VMEM_SHARED((NP,), jnp.float32),
            pltpu.SemaphoreType.DMA,
        ],
    )
    def deg_kernel(srcw_hbm, dstw_hbm, ones_hbm, zeros_hbm, out_hbm,
                   src_v, dst_v, ones_v, acc_o, acc_i, sem):
        c = lax.axis_index("c")
        s = lax.axis_index("s")
        wid = s * NC + c
        pltpu.sync_copy(zeros_hbm, acc_o.at[pl.ds(s * RPS, RPS)])
        pltpu.sync_copy(zeros_hbm, acc_i.at[pl.ds(s * RPS, RPS)])
        pltpu.sync_copy(srcw_hbm.at[wid], src_v)
        pltpu.sync_copy(dstw_hbm.at[wid], dst_v)
        pltpu.sync_copy(ones_hbm, ones_v)
        plsc.subcore_barrier()

        def step(i, carry):
            pltpu.async_copy(ones_v, acc_o.at[src_v.at[i]], sem, add=True)
            pltpu.async_copy(ones_v, acc_i.at[dst_v.at[i]], sem, add=True)
            return carry

        lax.fori_loop(0, CH, step, 0)

        def drain(i, carry):
            pltpu.make_async_copy(ones_hbm, ones_v, sem).wait()
            return carry

        lax.fori_loop(0, 2 * CH, drain, 0)
        plsc.subcore_barrier()
        pltpu.sync_copy(acc_o.at[pl.ds(s * RPS, RPS)],
                        out_hbm.at[c, 0, pl.ds(s * RPS, RPS)])
        pltpu.sync_copy(acc_i.at[pl.ds(s * RPS, RPS)],
                        out_hbm.at[c, 1, pl.ds(s * RPS, RPS)])

    return deg_kernel(srcw, dstw, ones_h, zeros_h)


def _sc_scatter(x, srcw, dstw, zeros_h):
    @functools.partial(
        pl.kernel,
        mesh=_mesh(),
        out_type=jax.ShapeDtypeStruct((NC, NP, D), jnp.float32),
        scratch_types=[
            pltpu.VMEM((2 * NB, CK), jnp.int32),
            pltpu.VMEM((2 * NB, CK), jnp.int32),
        ] + [pltpu.VMEM((CK, D), jnp.float32) for _ in range(NB)] + [
            pltpu.VMEM_SHARED((NP, D), jnp.float32),
        ] + [pltpu.SemaphoreType.DMA for _ in range(4 * NB)],
    )
    def scat_kernel(x_hbm, srcw_hbm, dstw_hbm, zeros_hbm, out_hbm,
                    src_v, dst_v, b0, b1, acc,
                    i0, i1, i2, i3, g0, g1, s0, s1):
        bufs = (b0, b1)
        isem = (i0, i1, i2, i3)
        gsem = (g0, g1)
        ssem = (s0, s1)
        c = lax.axis_index("c")
        s = lax.axis_index("s")
        wid = s * NC + c

        def load_idx(i, r):
            pltpu.async_copy(srcw_hbm.at[wid, i], src_v.at[r], isem[r])
            pltpu.async_copy(dstw_hbm.at[wid, i], dst_v.at[r], isem[r])

        def wait_idx(r):
            pltpu.make_async_copy(srcw_hbm.at[0, 0], src_v.at[r],
                                  isem[r]).wait()
            pltpu.make_async_copy(dstw_hbm.at[0, 0], dst_v.at[r],
                                  isem[r]).wait()

        def gather(r, b):
            pltpu.async_copy(x_hbm.at[src_v.at[r]], bufs[b], gsem[b])

        def scatter(r, b):
            pltpu.async_copy(bufs[b], acc.at[dst_v.at[r]], ssem[b], add=True)

        def wait_rows(b, sem):
            pltpu.make_async_copy(x_hbm.at[pl.ds(0, CK)], bufs[b],
                                  sem[b]).wait()

        for r in range(4):
            load_idx(r, r)
        for z in range(RPS // ZR):
            pltpu.sync_copy(zeros_hbm, acc.at[pl.ds(s * RPS + z * ZR, ZR)])
        plsc.subcore_barrier()
        for b in range(NB):
            wait_idx(b)
            gather(b, b)

        def sgroup(sg, carry):
            base = 4 * sg
            for b in range(NB):
                wait_rows(b, gsem)
                scatter(b, b)
            for b in range(NB):
                wait_idx(b + 2)
                wait_rows(b, ssem)
                gather(b + 2, b)

                @pl.when(base + 4 + b < CH)
                def _():
                    load_idx(base + 4 + b, b)
            for b in range(NB):
                wait_rows(b, gsem)
                scatter(b + 2, b)
            for b in range(NB):
                wait_rows(b, ssem)

                @pl.when(base + 4 + b < CH)
                def _():
                    wait_idx(b)
                    gather(b, b)

                @pl.when(base + 6 + b < CH)
                def _():
                    load_idx(base + 6 + b, b + 2)

            return carry

        lax.fori_loop(0, CH // 4, sgroup, 0)
        plsc.subcore_barrier()
        pltpu.sync_copy(acc.at[pl.ds(s * RPS, RPS)],
                        out_hbm.at[c, pl.ds(s * RPS, RPS)])

    return scat_kernel(x, srcw, dstw, zeros_h)


def _scales(deg_ref):
    s_out = lax.rsqrt(jnp.maximum(deg_ref[0, 0] + deg_ref[1, 0], 1.0))
    s_in = lax.rsqrt(jnp.maximum(deg_ref[0, 1] + deg_ref[1, 1], 1.0))
    return s_out, s_in


def _dot(a, b):
    return jnp.dot(a, b, preferred_element_type=jnp.float32)


def _tc_in_body(deg_ref, x_ref, win_ref, bin_ref, wc_ref, h_ref, x0_ref):
    s_out, _ = _scales(deg_ref)
    h = jnp.maximum(_dot(x_ref[...], win_ref[...]) + bin_ref[...], 0.0)
    h_ref[...] = h
    x0_ref[...] = _dot(h * s_out, wc_ref[...])


def _tc_gate_body(deg_ref, h_ref, p_ref, bc_ref, wg_ref, bg_ref, wc_ref,
                  h1_ref, x1_ref):
    s_out, s_in = _scales(deg_ref)
    cell = (p_ref[0] + p_ref[1]) * s_in + bc_ref[...]
    h = h_ref[...]
    wg = wg_ref[...]
    cc = jax.nn.sigmoid(_dot(h, wg[0:D]) + _dot(cell, wg[D:2 * D])
                        + bg_ref[...])
    h1_ref[...] = h * cc[:, 0:1] + cell * cc[:, 1:2]
    x1_ref[...] = _dot(cell * cc[:, 3:4] * s_out, wc_ref[...])


def _tc_out_body(deg_ref, h_ref, p_ref, bc_ref, wg_ref, bg_ref, wcls_ref,
                 bcls_ref, out_ref):
    _, s_in = _scales(deg_ref)
    cell = (p_ref[0] + p_ref[1]) * s_in + bc_ref[...]
    h = h_ref[...]
    wg = wg_ref[...]
    cc = jax.nn.sigmoid(_dot(h, wg[0:D]) + _dot(cell, wg[D:2 * D])
                        + bg_ref[...])
    h2 = h * cc[:, 0:1] + cell * cc[:, 1:2]
    out_ref[...] = _dot(h2, wcls_ref[...]) + bcls_ref[...]


def _spec_deg():
    return pl.BlockSpec((NC, 2, R, 1), lambda i: (0, 0, i, 0))


def _spec_rows():
    return pl.BlockSpec((R, D), lambda i: (i, 0))


def _spec_full(shape):
    nd = len(shape)
    return pl.BlockSpec(shape, lambda i: (0,) * nd)


def _tc_in(degp, x, w_in, b_in, w_conv):
    return pl.pallas_call(
        _tc_in_body,
        grid=(N // R,),
        in_specs=[
            _spec_deg(), _spec_rows(),
            _spec_full((D, D)), _spec_full((1, D)), _spec_full((D, D)),
        ],
        out_specs=[_spec_rows(), _spec_rows()],
        out_shape=[jax.ShapeDtypeStruct((N, D), jnp.float32),
                   jax.ShapeDtypeStruct((N, D), jnp.float32)],
    )(degp, x, w_in, b_in, w_conv)


def _tc_gate(degp, h, p, b_conv, w_gate, b_gate, w_conv_next):
    return pl.pallas_call(
        _tc_gate_body,
        grid=(N // R,),
        in_specs=[
            _spec_deg(), _spec_rows(),
            pl.BlockSpec((NC, R, D), lambda i: (0, i, 0)),
            _spec_full((1, D)), _spec_full((2 * D, 4)), _spec_full((1, 4)),
            _spec_full((D, D)),
        ],
        out_specs=[_spec_rows(), _spec_rows()],
        out_shape=[jax.ShapeDtypeStruct((N, D), jnp.float32),
                   jax.ShapeDtypeStruct((N, D), jnp.float32)],
    )(degp, h, p, b_conv, w_gate, b_gate, w_conv_next)


def _tc_out(degp, h, p, b_conv, w_gate, b_gate, w_cls, b_cls):
    return pl.pallas_call(
        _tc_out_body,
        grid=(N // R,),
        in_specs=[
            _spec_deg(), _spec_rows(),
            pl.BlockSpec((NC, R, D), lambda i: (0, i, 0)),
            _spec_full((1, D)), _spec_full((2 * D, 4)), _spec_full((1, 4)),
            _spec_full((D, D_OUT)), _spec_full((1, D_OUT)),
        ],
        out_specs=pl.BlockSpec((R, D_OUT), lambda i: (i, 0)),
        out_shape=jax.ShapeDtypeStruct((N, D_OUT), jnp.float32),
    )(degp, h, p, b_conv, w_gate, b_gate, w_cls, b_cls)


def kernel(inputs, graph, W_in, b_in, W_conv0, b_conv0, W_conv1, b_conv1,
           W_gate, b_gate, W_cls, b_cls):
    pad = EP - E
    srcw = jnp.concatenate(
        [graph[0], jnp.zeros((pad,), jnp.int32)]).reshape(NW, CH, CK)
    dstw = jnp.concatenate(
        [graph[1], jnp.full((pad,), DUMP, jnp.int32)]).reshape(NW, CH, CK)
    ones_h = jnp.ones((CK,), jnp.float32)
    zeros_deg = jnp.zeros((RPS,), jnp.float32)
    zeros_row = jnp.zeros((ZR, D), jnp.float32)

    degp = _sc_degrees(srcw, dstw, ones_h, zeros_deg).reshape(NC, 2, NP, 1)

    h, x0 = _tc_in(degp, inputs, W_in, b_in.reshape(1, D), W_conv0)
    p0 = _sc_scatter(x0, srcw, dstw, zeros_row)
    h1, x1 = _tc_gate(degp, h, p0, b_conv0.reshape(1, D), W_gate,
                      b_gate.reshape(1, 4), W_conv1)
    p1 = _sc_scatter(x1, srcw, dstw, zeros_row)
    out = _tc_out(degp, h1, p1, b_conv1.reshape(1, D), W_gate,
                  b_gate.reshape(1, 4), W_cls, b_cls.reshape(1, D_OUT))
    return out


# speedup vs baseline: 1.0192x; 1.0192x over previous
import functools

import jax
import jax.numpy as jnp
from jax import lax
from jax.experimental import pallas as pl
from jax.experimental.pallas import tpu as pltpu
from jax.experimental.pallas import tpu_sc as plsc

N = 10000
NP = 10240
E = 320000
D = 128
D_OUT = 64

NC = 2
NS = 16
NW = NC * NS
CK = 128
CH = 80
EP = NW * CH * CK
DUMP = NP - 8
NB = 2
RPS = NP // NS
ZR = 128

R = 1000


def _mesh():
    return plsc.VectorSubcoreMesh(core_axis_name="c", subcore_axis_name="s")


def _sc_degrees(srcw, dstw, ones_h, zeros_h):
    @functools.partial(
        pl.kernel,
        mesh=_mesh(),
        out_type=jax.ShapeDtypeStruct((NC, 2, NP), jnp.float32),
        scratch_types=[
            pltpu.VMEM((CH, CK), jnp.int32),
            pltpu.VMEM((CH, CK), jnp.int32),
            pltpu.VMEM((CK,), jnp.float32),
            pltpu.VMEM_SHARED((NP,), jnp.float32),
            pltpu.VMEM_SHARED((NP,), jnp.float32),
            pltpu.SemaphoreType.DMA,
        ],
    )
    def deg_kernel(srcw_hbm, dstw_hbm, ones_hbm, zeros_hbm, out_hbm,
                   src_v, dst_v, ones_v, acc_o, acc_i, sem):
        c = lax.axis_index("c")
        s = lax.axis_index("s")
        wid = s * NC + c
        pltpu.sync_copy(zeros_hbm, acc_o.at[pl.ds(s * RPS, RPS)])
        pltpu.sync_copy(zeros_hbm, acc_i.at[pl.ds(s * RPS, RPS)])
        pltpu.sync_copy(srcw_hbm.at[wid], src_v)
        pltpu.sync_copy(dstw_hbm.at[wid], dst_v)
        pltpu.sync_copy(ones_hbm, ones_v)
        plsc.subcore_barrier()

        def step(i, carry):
            pltpu.async_copy(ones_v, acc_o.at[src_v.at[i]], sem, add=True)
            pltpu.async_copy(ones_v, acc_i.at[dst_v.at[i]], sem, add=True)
            return carry

        lax.fori_loop(0, CH, step, 0)

        def drain(i, carry):
            pltpu.make_async_copy(ones_hbm, ones_v, sem).wait()
            return carry

        lax.fori_loop(0, 2 * CH, drain, 0)
        plsc.subcore_barrier()
        pltpu.sync_copy(acc_o.at[pl.ds(s * RPS, RPS)],
                        out_hbm.at[c, 0, pl.ds(s * RPS, RPS)])
        pltpu.sync_copy(acc_i.at[pl.ds(s * RPS, RPS)],
                        out_hbm.at[c, 1, pl.ds(s * RPS, RPS)])

    return deg_kernel(srcw, dstw, ones_h, zeros_h)


def _sc_scatter(x, srcw, dstw, zeros_h):
    @functools.partial(
        pl.kernel,
        mesh=_mesh(),
        out_type=jax.ShapeDtypeStruct((NC, NP, D), jnp.float32),
        scratch_types=[
            pltpu.VMEM((2 * NB, CK), jnp.int32),
            pltpu.VMEM((2 * NB, CK), jnp.int32),
        ] + [pltpu.VMEM((CK, D), jnp.float32) for _ in range(NB)] + [
            pltpu.VMEM_SHARED((NP, D), jnp.float32),
        ] + [pltpu.SemaphoreType.DMA for _ in range(4 * NB)],
    )
    def scat_kernel(x_hbm, srcw_hbm, dstw_hbm, zeros_hbm, out_hbm,
                    src_v, dst_v, b0, b1, acc,
                    i0, i1, i2, i3, g0, g1, s0, s1):
        bufs = (b0, b1)
        isem = (i0, i1, i2, i3)
        gsem = (g0, g1)
        ssem = (s0, s1)
        c = lax.axis_index("c")
        s = lax.axis_index("s")
        wid = s * NC + c

        def load_idx(i, r):
            pltpu.async_copy(srcw_hbm.at[wid, i], src_v.at[r], isem[r])
            pltpu.async_copy(dstw_hbm.at[wid, i], dst_v.at[r], isem[r])

        def wait_idx(r):
            pltpu.make_async_copy(srcw_hbm.at[0, 0], src_v.at[r],
                                  isem[r]).wait()
            pltpu.make_async_copy(dstw_hbm.at[0, 0], dst_v.at[r],
                                  isem[r]).wait()

        def gather(r, b):
            pltpu.async_copy(x_hbm.at[src_v.at[r]], bufs[b], gsem[b])

        def scatter(r, b):
            pltpu.async_copy(bufs[b], acc.at[dst_v.at[r]], ssem[b], add=True)

        def wait_rows(b, sem):
            pltpu.make_async_copy(x_hbm.at[pl.ds(0, CK)], bufs[b],
                                  sem[b]).wait()

        for r in range(4):
            load_idx(r, r)
        for z in range(RPS // ZR):
            pltpu.sync_copy(zeros_hbm, acc.at[pl.ds(s * RPS + z * ZR, ZR)])
        plsc.subcore_barrier()
        for b in range(NB):
            wait_idx(b)
            gather(b, b)

        def sgroup(sg, carry):
            base = 4 * sg
            for b in range(NB):
                wait_rows(b, gsem)
                scatter(b, b)
            for b in range(NB):
                wait_idx(b + 2)
                wait_rows(b, ssem)
                gather(b + 2, b)

                @pl.when(base + 4 + b < CH)
                def _():
                    load_idx(base + 4 + b, b)
            for b in range(NB):
                wait_rows(b, gsem)
                scatter(b + 2, b)
            for b in range(NB):
                wait_rows(b, ssem)

                @pl.when(base + 4 + b < CH)
                def _():
                    wait_idx(b)
                    gather(b, b)

                @pl.when(base + 6 + b < CH)
                def _():
                    load_idx(base + 6 + b, b + 2)

            return carry

        lax.fori_loop(0, CH // 4, sgroup, 0)
        plsc.subcore_barrier()
        pltpu.sync_copy(acc.at[pl.ds(s * RPS, RPS)],
                        out_hbm.at[c, pl.ds(s * RPS, RPS)])

    return scat_kernel(x, srcw, dstw, zeros_h)


def _scales(deg_ref):
    s_out = lax.rsqrt(jnp.maximum(deg_ref[0, 0] + deg_ref[1, 0], 1.0))
    s_in = lax.rsqrt(jnp.maximum(deg_ref[0, 1] + deg_ref[1, 1], 1.0))
    return s_out, s_in


def _dot(a, b):
    return jnp.dot(a, b, preferred_element_type=jnp.float32)


def _tc_in_body(deg_ref, x_ref, win_ref, bin_ref, wc_ref, h_ref, x0_ref):
    s_out, _ = _scales(deg_ref)
    h = jnp.maximum(_dot(x_ref[...], win_ref[...]) + bin_ref[...], 0.0)
    h_ref[...] = h
    x0_ref[...] = _dot(h * s_out, wc_ref[...])


def _tc_gate_body(deg_ref, h_ref, p_ref, bc_ref, wg_ref, bg_ref, wc_ref,
                  h1_ref, x1_ref):
    s_out, s_in = _scales(deg_ref)
    cell = (p_ref[0] + p_ref[1]) * s_in + bc_ref[...]
    h = h_ref[...]
    wg = wg_ref[...]
    cc = jax.nn.sigmoid(_dot(h, wg[0:D]) + _dot(cell, wg[D:2 * D])
                        + bg_ref[...])
    h1_ref[...] = h * cc[:, 0:1] + cell * cc[:, 1:2]
    x1_ref[...] = _dot(cell * cc[:, 3:4] * s_out, wc_ref[...])


def _tc_out_body(deg_ref, h_ref, p_ref, bc_ref, wg_ref, bg_ref, wcls_ref,
                 bcls_ref, out_ref):
    _, s_in = _scales(deg_ref)
    cell = (p_ref[0] + p_ref[1]) * s_in + bc_ref[...]
    h = h_ref[...]
    wg = wg_ref[...]
    cc = jax.nn.sigmoid(_dot(h, wg[0:D]) + _dot(cell, wg[D:2 * D])
                        + bg_ref[...])
    h2 = h * cc[:, 0:1] + cell * cc[:, 1:2]
    out_ref[...] = _dot(h2, wcls_ref[...]) + bcls_ref[...]


def _spec_deg():
    return pl.BlockSpec((NC, 2, R, 1), lambda i: (0, 0, i, 0))


def _spec_rows():
    return pl.BlockSpec((R, D), lambda i: (i, 0))


def _spec_full(shape):
    nd = len(shape)
    return pl.BlockSpec(shape, lambda i: (0,) * nd)


def _tc_in(degp, x, w_in, b_in, w_conv):
    return pl.pallas_call(
        _tc_in_body,
        grid=(N // R,),
        in_specs=[
            _spec_deg(), _spec_rows(),
            _spec_full((D, D)), _spec_full((1, D)), _spec_full((D, D)),
        ],
        out_specs=[_spec_rows(), _spec_rows()],
        out_shape=[jax.ShapeDtypeStruct((N, D), jnp.float32),
                   jax.ShapeDtypeStruct((N, D), jnp.float32)],
    )(degp, x, w_in, b_in, w_conv)


def _tc_gate(degp, h, p, b_conv, w_gate, b_gate, w_conv_next):
    return pl.pallas_call(
        _tc_gate_body,
        grid=(N // R,),
        in_specs=[
            _spec_deg(), _spec_rows(),
            pl.BlockSpec((NC, R, D), lambda i: (0, i, 0)),
            _spec_full((1, D)), _spec_full((2 * D, 4)), _spec_full((1, 4)),
            _spec_full((D, D)),
        ],
        out_specs=[_spec_rows(), _spec_rows()],
        out_shape=[jax.ShapeDtypeStruct((N, D), jnp.float32),
                   jax.ShapeDtypeStruct((N, D), jnp.float32)],
    )(degp, h, p, b_conv, w_gate, b_gate, w_conv_next)


def _tc_out(degp, h, p, b_conv, w_gate, b_gate, w_cls, b_cls):
    return pl.pallas_call(
        _tc_out_body,
        grid=(N // R,),
        in_specs=[
            _spec_deg(), _spec_rows(),
            pl.BlockSpec((NC, R, D), lambda i: (0, i, 0)),
            _spec_full((1, D)), _spec_full((2 * D, 4)), _spec_full((1, 4)),
            _spec_full((D, D_OUT)), _spec_full((1, D_OUT)),
        ],
        out_specs=pl.BlockSpec((R, D_OUT), lambda i: (i, 0)),
        out_shape=jax.ShapeDtypeStruct((N, D_OUT), jnp.float32),
    )(degp, h, p, b_conv, w_gate, b_gate, w_cls, b_cls)


def kernel(inputs, graph, W_in, b_in, W_conv0, b_conv0, W_conv1, b_conv1,
           W_gate, b_gate, W_cls, b_cls):
    padw = CH * CK - E // NW
    srcw = jnp.pad(graph[0].reshape(NW, E // NW),
                   ((0, 0), (0, padw))).reshape(NW, CH, CK)
    dummy = jnp.arange(padw, dtype=jnp.int32) % (NP - N) + N
    dstw = jnp.concatenate(
        [graph[1].reshape(NW, E // NW),
         jnp.broadcast_to(dummy, (NW, padw))], axis=1).reshape(NW, CH, CK)
    ones_h = jnp.ones((CK,), jnp.float32)
    zeros_deg = jnp.zeros((RPS,), jnp.float32)
    zeros_row = jnp.zeros((ZR, D), jnp.float32)

    degp = _sc_degrees(srcw, dstw, ones_h, zeros_deg).reshape(NC, 2, NP, 1)

    h, x0 = _tc_in(degp, inputs, W_in, b_in.reshape(1, D), W_conv0)
    p0 = _sc_scatter(x0, srcw, dstw, zeros_row)
    h1, x1 = _tc_gate(degp, h, p0, b_conv0.reshape(1, D), W_gate,
                      b_gate.reshape(1, 4), W_conv1)
    p1 = _sc_scatter(x1, srcw, dstw, zeros_row)
    out = _tc_out(degp, h1, p1, b_conv1.reshape(1, D), W_gate,
                  b_gate.reshape(1, 4), W_cls, b_cls.reshape(1, D_OUT))
    return out


# speedup vs baseline: 1.0970x; 1.0764x over previous
import functools

import jax
import jax.numpy as jnp
from jax import lax
from jax.experimental import pallas as pl
from jax.experimental.pallas import tpu as pltpu
from jax.experimental.pallas import tpu_sc as plsc

N = 10000
NP = 10240
E = 320000
D = 128
D_OUT = 64

NC = 2
NS = 16
NW = NC * NS
CK = 128
CH = 80
EP = NW * CH * CK
DUMP = NP - 8
NB = 2
RPS = NP // NS
ZR = 128

R = 1000


def _mesh():
    return plsc.VectorSubcoreMesh(core_axis_name="c", subcore_axis_name="s")


def _sc_degrees(srcw, dstw, ones_h, zeros_h):
    @functools.partial(
        pl.kernel,
        mesh=_mesh(),
        out_type=jax.ShapeDtypeStruct((NC, 2, NP), jnp.float32),
        scratch_types=[
            pltpu.VMEM((CH, CK), jnp.int32),
            pltpu.VMEM((CH, CK), jnp.int32),
            pltpu.VMEM((CK,), jnp.float32),
            pltpu.VMEM_SHARED((NP,), jnp.float32),
            pltpu.VMEM_SHARED((NP,), jnp.float32),
            pltpu.SemaphoreType.DMA,
        ],
    )
    def deg_kernel(srcw_hbm, dstw_hbm, ones_hbm, zeros_hbm, out_hbm,
                   src_v, dst_v, ones_v, acc_o, acc_i, sem):
        c = lax.axis_index("c")
        s = lax.axis_index("s")
        wid = s * NC + c
        pltpu.sync_copy(zeros_hbm, acc_o.at[pl.ds(s * RPS, RPS)])
        pltpu.sync_copy(zeros_hbm, acc_i.at[pl.ds(s * RPS, RPS)])
        pltpu.sync_copy(srcw_hbm.at[wid], src_v)
        pltpu.sync_copy(dstw_hbm.at[wid], dst_v)
        pltpu.sync_copy(ones_hbm, ones_v)
        plsc.subcore_barrier()

        def step(i, carry):
            pltpu.async_copy(ones_v, acc_o.at[src_v.at[i]], sem, add=True)
            pltpu.async_copy(ones_v, acc_i.at[dst_v.at[i]], sem, add=True)
            return carry

        lax.fori_loop(0, CH, step, 0)

        def drain(i, carry):
            pltpu.make_async_copy(ones_hbm, ones_v, sem).wait()
            return carry

        lax.fori_loop(0, 2 * CH, drain, 0)
        plsc.subcore_barrier()
        pltpu.sync_copy(acc_o.at[pl.ds(s * RPS, RPS)],
                        out_hbm.at[c, 0, pl.ds(s * RPS, RPS)])
        pltpu.sync_copy(acc_i.at[pl.ds(s * RPS, RPS)],
                        out_hbm.at[c, 1, pl.ds(s * RPS, RPS)])

    return deg_kernel(srcw, dstw, ones_h, zeros_h)


def _sc_scatter(x, srcw, dstw, zeros_h):
    @functools.partial(
        pl.kernel,
        mesh=_mesh(),
        out_type=jax.ShapeDtypeStruct((NC, NP, D), jnp.float32),
        scratch_types=[
            pltpu.VMEM((2 * NB, CK), jnp.int32),
            pltpu.VMEM((2 * NB, CK), jnp.int32),
        ] + [pltpu.VMEM((CK, D), jnp.float32) for _ in range(NB)] + [
            pltpu.VMEM_SHARED((NP, D), jnp.float32),
        ] + [pltpu.SemaphoreType.DMA for _ in range(3 * NB)],
    )
    def scat_kernel(x_hbm, srcw_hbm, dstw_hbm, zeros_hbm, out_hbm,
                    src_v, dst_v, b0, b1, acc,
                    i0, i1, i2, i3, g0, g1):
        bufs = (b0, b1)
        isem = (i0, i1, i2, i3)
        gsem = (g0, g1)
        c = lax.axis_index("c")
        s = lax.axis_index("s")
        wid = s * NC + c

        def load_idx(i, r):
            pltpu.async_copy(srcw_hbm.at[wid, i], src_v.at[r], isem[r])
            pltpu.async_copy(dstw_hbm.at[wid, i], dst_v.at[r], isem[r])

        def wait_idx(r):
            pltpu.make_async_copy(srcw_hbm.at[0, 0], src_v.at[r],
                                  isem[r]).wait()
            pltpu.make_async_copy(dstw_hbm.at[0, 0], dst_v.at[r],
                                  isem[r]).wait()

        def gather(r, b):
            pltpu.async_copy(x_hbm.at[src_v.at[r]], bufs[b], gsem[b])

        def wait_rows(b):
            pltpu.make_async_copy(x_hbm.at[pl.ds(0, CK)], bufs[b],
                                  gsem[b]).wait()

        for r in range(4):
            load_idx(r, r)
        for z in range(RPS // ZR):
            pltpu.sync_copy(zeros_hbm, acc.at[pl.ds(s * RPS + z * ZR, ZR)])
        plsc.subcore_barrier()
        for b in range(NB):
            wait_idx(b)
            gather(b, b)

        def sgroup(sg, carry):
            base = 4 * sg
            for t in range(4):
                b = t % 2
                i = base + t
                wait_rows(b)
                pltpu.sync_copy(bufs[b], acc.at[dst_v.at[t]], add=True)
                if t < 2:
                    wait_idx(t + 2)
                    gather(t + 2, b)
                else:

                    @pl.when(i + 2 < CH)
                    def _():
                        wait_idx(t - 2)
                        gather(t - 2, b)

                @pl.when(i + 4 < CH)
                def _():
                    load_idx(i + 4, t)

            return carry

        lax.fori_loop(0, CH // 4, sgroup, 0)
        plsc.subcore_barrier()
        pltpu.sync_copy(acc.at[pl.ds(s * RPS, RPS)],
                        out_hbm.at[c, pl.ds(s * RPS, RPS)])

    return scat_kernel(x, srcw, dstw, zeros_h)


def _scales(deg_ref):
    s_out = lax.rsqrt(jnp.maximum(deg_ref[0, 0] + deg_ref[1, 0], 1.0))
    s_in = lax.rsqrt(jnp.maximum(deg_ref[0, 1] + deg_ref[1, 1], 1.0))
    return s_out, s_in


def _dot(a, b):
    return jnp.dot(a, b, preferred_element_type=jnp.float32)


def _tc_in_body(deg_ref, x_ref, win_ref, bin_ref, wc_ref, h_ref, x0_ref):
    s_out, _ = _scales(deg_ref)
    h = jnp.maximum(_dot(x_ref[...], win_ref[...]) + bin_ref[...], 0.0)
    h_ref[...] = h
    x0_ref[...] = _dot(h * s_out, wc_ref[...])


def _tc_gate_body(deg_ref, h_ref, p_ref, bc_ref, wg_ref, bg_ref, wc_ref,
                  h1_ref, x1_ref):
    s_out, s_in = _scales(deg_ref)
    cell = (p_ref[0] + p_ref[1]) * s_in + bc_ref[...]
    h = h_ref[...]
    wg = wg_ref[...]
    cc = jax.nn.sigmoid(_dot(h, wg[0:D]) + _dot(cell, wg[D:2 * D])
                        + bg_ref[...])
    h1_ref[...] = h * cc[:, 0:1] + cell * cc[:, 1:2]
    x1_ref[...] = _dot(cell * cc[:, 3:4] * s_out, wc_ref[...])


def _tc_out_body(deg_ref, h_ref, p_ref, bc_ref, wg_ref, bg_ref, wcls_ref,
                 bcls_ref, out_ref):
    _, s_in = _scales(deg_ref)
    cell = (p_ref[0] + p_ref[1]) * s_in + bc_ref[...]
    h = h_ref[...]
    wg = wg_ref[...]
    cc = jax.nn.sigmoid(_dot(h, wg[0:D]) + _dot(cell, wg[D:2 * D])
                        + bg_ref[...])
    h2 = h * cc[:, 0:1] + cell * cc[:, 1:2]
    out_ref[...] = _dot(h2, wcls_ref[...]) + bcls_ref[...]


def _spec_deg():
    return pl.BlockSpec((NC, 2, R, 1), lambda i: (0, 0, i, 0))


def _spec_rows():
    return pl.BlockSpec((R, D), lambda i: (i, 0))


def _spec_full(shape):
    nd = len(shape)
    return pl.BlockSpec(shape, lambda i: (0,) * nd)


def _tc_in(degp, x, w_in, b_in, w_conv):
    return pl.pallas_call(
        _tc_in_body,
        grid=(N // R,),
        in_specs=[
            _spec_deg(), _spec_rows(),
            _spec_full((D, D)), _spec_full((1, D)), _spec_full((D, D)),
        ],
        out_specs=[_spec_rows(), _spec_rows()],
        out_shape=[jax.ShapeDtypeStruct((N, D), jnp.float32),
                   jax.ShapeDtypeStruct((N, D), jnp.float32)],
    )(degp, x, w_in, b_in, w_conv)


def _tc_gate(degp, h, p, b_conv, w_gate, b_gate, w_conv_next):
    return pl.pallas_call(
        _tc_gate_body,
        grid=(N // R,),
        in_specs=[
            _spec_deg(), _spec_rows(),
            pl.BlockSpec((NC, R, D), lambda i: (0, i, 0)),
            _spec_full((1, D)), _spec_full((2 * D, 4)), _spec_full((1, 4)),
            _spec_full((D, D)),
        ],
        out_specs=[_spec_rows(), _spec_rows()],
        out_shape=[jax.ShapeDtypeStruct((N, D), jnp.float32),
                   jax.ShapeDtypeStruct((N, D), jnp.float32)],
    )(degp, h, p, b_conv, w_gate, b_gate, w_conv_next)


def _tc_out(degp, h, p, b_conv, w_gate, b_gate, w_cls, b_cls):
    return pl.pallas_call(
        _tc_out_body,
        grid=(N // R,),
        in_specs=[
            _spec_deg(), _spec_rows(),
            pl.BlockSpec((NC, R, D), lambda i: (0, i, 0)),
            _spec_full((1, D)), _spec_full((2 * D, 4)), _spec_full((1, 4)),
            _spec_full((D, D_OUT)), _spec_full((1, D_OUT)),
        ],
        out_specs=pl.BlockSpec((R, D_OUT), lambda i: (i, 0)),
        out_shape=jax.ShapeDtypeStruct((N, D_OUT), jnp.float32),
    )(degp, h, p, b_conv, w_gate, b_gate, w_cls, b_cls)


def kernel(inputs, graph, W_in, b_in, W_conv0, b_conv0, W_conv1, b_conv1,
           W_gate, b_gate, W_cls, b_cls):
    padw = CH * CK - E // NW
    srcw = jnp.pad(graph[0].reshape(NW, E // NW),
                   ((0, 0), (0, padw))).reshape(NW, CH, CK)
    dummy = jnp.arange(padw, dtype=jnp.int32) % (NP - N) + N
    dstw = jnp.concatenate(
        [graph[1].reshape(NW, E // NW),
         jnp.broadcast_to(dummy, (NW, padw))], axis=1).reshape(NW, CH, CK)
    ones_h = jnp.ones((CK,), jnp.float32)
    zeros_deg = jnp.zeros((RPS,), jnp.float32)
    zeros_row = jnp.zeros((ZR, D), jnp.float32)

    degp = _sc_degrees(srcw, dstw, ones_h, zeros_deg).reshape(NC, 2, NP, 1)

    h, x0 = _tc_in(degp, inputs, W_in, b_in.reshape(1, D), W_conv0)
    p0 = _sc_scatter(x0, srcw, dstw, zeros_row)
    h1, x1 = _tc_gate(degp, h, p0, b_conv0.reshape(1, D), W_gate,
                      b_gate.reshape(1, 4), W_conv1)
    p1 = _sc_scatter(x1, srcw, dstw, zeros_row)
    out = _tc_out(degp, h1, p1, b_conv1.reshape(1, D), W_gate,
                  b_gate.reshape(1, 4), W_cls, b_cls.reshape(1, D_OUT))
    return out


# speedup vs baseline: 2.0018x; 1.8248x over previous
import functools

import jax
import jax.numpy as jnp
from jax import lax
from jax.experimental import pallas as pl
from jax.experimental.pallas import tpu as pltpu
from jax.experimental.pallas import tpu_sc as plsc

N = 10000
NP = 10240
E = 320000
D = 128
D_OUT = 64

NC = 2
NS = 16
NW = NC * NS
CK = 128
CH = 80
EPW = E // NW
K = 80
EP = NW * CH * CK
DUMP = NP - 8
NB = 2
RPS = NP // NS
ZR = 128

R = 1000


def _mesh():
    return plsc.VectorSubcoreMesh(core_axis_name="c", subcore_axis_name="s")


def _sc_degrees(srcw, dstw, ones_h, zeros_h):
    @functools.partial(
        pl.kernel,
        mesh=_mesh(),
        out_type=jax.ShapeDtypeStruct((NC, 2, NP), jnp.float32),
        scratch_types=[
            pltpu.VMEM((CH, CK), jnp.int32),
            pltpu.VMEM((CH, CK), jnp.int32),
            pltpu.VMEM((CK,), jnp.float32),
            pltpu.VMEM_SHARED((NP,), jnp.float32),
            pltpu.VMEM_SHARED((NP,), jnp.float32),
            pltpu.SemaphoreType.DMA,
        ],
    )
    def deg_kernel(srcw_hbm, dstw_hbm, ones_hbm, zeros_hbm, out_hbm,
                   src_v, dst_v, ones_v, acc_o, acc_i, sem):
        c = lax.axis_index("c")
        s = lax.axis_index("s")
        wid = s * NC + c
        pltpu.sync_copy(zeros_hbm, acc_o.at[pl.ds(s * RPS, RPS)])
        pltpu.sync_copy(zeros_hbm, acc_i.at[pl.ds(s * RPS, RPS)])
        pltpu.sync_copy(srcw_hbm.at[wid], src_v)
        pltpu.sync_copy(dstw_hbm.at[wid], dst_v)
        pltpu.sync_copy(ones_hbm, ones_v)
        plsc.subcore_barrier()

        def step(i, carry):
            pltpu.async_copy(ones_v, acc_o.at[src_v.at[i]], sem, add=True)
            pltpu.async_copy(ones_v, acc_i.at[dst_v.at[i]], sem, add=True)
            return carry

        lax.fori_loop(0, CH, step, 0)

        def drain(i, carry):
            pltpu.make_async_copy(ones_hbm, ones_v, sem).wait()
            return carry

        lax.fori_loop(0, 2 * CH, drain, 0)
        plsc.subcore_barrier()
        pltpu.sync_copy(acc_o.at[pl.ds(s * RPS, RPS)],
                        out_hbm.at[c, 0, pl.ds(s * RPS, RPS)])
        pltpu.sync_copy(acc_i.at[pl.ds(s * RPS, RPS)],
                        out_hbm.at[c, 1, pl.ds(s * RPS, RPS)])

    return deg_kernel(srcw, dstw, ones_h, zeros_h)


def _sc_scatter(x, src, dst, zeros_h):
    nch = EPW // K

    @functools.partial(
        pl.kernel,
        mesh=_mesh(),
        out_type=jax.ShapeDtypeStruct((NC, NP, D), jnp.float32),
        scratch_types=[
            pltpu.VMEM((K,), jnp.int32),
            pltpu.VMEM((K,), jnp.int32),
            pltpu.VMEM((K,), jnp.int32),
            pltpu.VMEM((K,), jnp.int32),
            pltpu.VMEM((K, D), jnp.float32),
            pltpu.VMEM((K, D), jnp.float32),
            pltpu.VMEM_SHARED((NP, D), jnp.float32),
            pltpu.SemaphoreType.DMA,
            pltpu.SemaphoreType.DMA,
        ],
    )
    def scat_kernel(x_hbm, src_hbm, dst_hbm, zeros_hbm, out_hbm,
                    sa, da, sb, db, bufa, bufb, acc, sema, semb):
        c = lax.axis_index("c")
        s = lax.axis_index("s")
        wid = s * NC + c
        base = wid * EPW

        def load_idx(i, sv, dv):
            off = base + i * K
            pltpu.sync_copy(src_hbm.at[pl.ds(off, K)], sv)
            pltpu.sync_copy(dst_hbm.at[pl.ds(off, K)], dv)

        def wait_rows(buf, sem):
            pltpu.make_async_copy(x_hbm.at[pl.ds(0, K)], buf, sem).wait()

        for z in range(RPS // ZR):
            pltpu.sync_copy(zeros_hbm, acc.at[pl.ds(s * RPS + z * ZR, ZR)])
        load_idx(0, sa, da)
        pltpu.async_copy(x_hbm.at[sa], bufa, sema)
        load_idx(1, sb, db)
        pltpu.async_copy(x_hbm.at[sb], bufb, semb)
        plsc.subcore_barrier()

        def pair(g, carry):
            wait_rows(bufa, sema)
            pltpu.sync_copy(bufa, acc.at[da], add=True)
            load_idx(2 * g + 2, sa, da)
            pltpu.async_copy(x_hbm.at[sa], bufa, sema)
            wait_rows(bufb, semb)
            pltpu.sync_copy(bufb, acc.at[db], add=True)

            @pl.when(2 * g + 3 < nch)
            def _():
                load_idx(2 * g + 3, sb, db)
                pltpu.async_copy(x_hbm.at[sb], bufb, semb)

            return carry

        lax.fori_loop(0, (nch - 1) // 2, pair, 0)
        wait_rows(bufa, sema)
        pltpu.sync_copy(bufa, acc.at[da], add=True)
        plsc.subcore_barrier()
        pltpu.sync_copy(acc.at[pl.ds(s * RPS, RPS)],
                        out_hbm.at[c, pl.ds(s * RPS, RPS)])

    return scat_kernel(x, src, dst, zeros_h)


def _scales(deg_ref):
    s_out = lax.rsqrt(jnp.maximum(deg_ref[0, 0] + deg_ref[1, 0], 1.0))
    s_in = lax.rsqrt(jnp.maximum(deg_ref[0, 1] + deg_ref[1, 1], 1.0))
    return s_out, s_in


def _dot(a, b):
    return jnp.dot(a, b, preferred_element_type=jnp.float32)


def _tc_in_body(deg_ref, x_ref, win_ref, bin_ref, wc_ref, h_ref, x0_ref):
    s_out, _ = _scales(deg_ref)
    h = jnp.maximum(_dot(x_ref[...], win_ref[...]) + bin_ref[...], 0.0)
    h_ref[...] = h
    x0_ref[...] = _dot(h * s_out, wc_ref[...])


def _tc_gate_body(deg_ref, h_ref, p_ref, bc_ref, wg_ref, bg_ref, wc_ref,
                  h1_ref, x1_ref):
    s_out, s_in = _scales(deg_ref)
    cell = (p_ref[0] + p_ref[1]) * s_in + bc_ref[...]
    h = h_ref[...]
    wg = wg_ref[...]
    cc = jax.nn.sigmoid(_dot(h, wg[0:D]) + _dot(cell, wg[D:2 * D])
                        + bg_ref[...])
    h1_ref[...] = h * cc[:, 0:1] + cell * cc[:, 1:2]
    x1_ref[...] = _dot(cell * cc[:, 3:4] * s_out, wc_ref[...])


def _tc_out_body(deg_ref, h_ref, p_ref, bc_ref, wg_ref, bg_ref, wcls_ref,
                 bcls_ref, out_ref):
    _, s_in = _scales(deg_ref)
    cell = (p_ref[0] + p_ref[1]) * s_in + bc_ref[...]
    h = h_ref[...]
    wg = wg_ref[...]
    cc = jax.nn.sigmoid(_dot(h, wg[0:D]) + _dot(cell, wg[D:2 * D])
                        + bg_ref[...])
    h2 = h * cc[:, 0:1] + cell * cc[:, 1:2]
    out_ref[...] = _dot(h2, wcls_ref[...]) + bcls_ref[...]


def _spec_deg():
    return pl.BlockSpec((NC, 2, R, 1), lambda i: (0, 0, i, 0))


def _spec_rows():
    return pl.BlockSpec((R, D), lambda i: (i, 0))


def _spec_full(shape):
    nd = len(shape)
    return pl.BlockSpec(shape, lambda i: (0,) * nd)


def _tc_in(degp, x, w_in, b_in, w_conv):
    return pl.pallas_call(
        _tc_in_body,
        grid=(N // R,),
        in_specs=[
            _spec_deg(), _spec_rows(),
            _spec_full((D, D)), _spec_full((1, D)), _spec_full((D, D)),
        ],
        out_specs=[_spec_rows(), _spec_rows()],
        out_shape=[jax.ShapeDtypeStruct((N, D), jnp.float32),
                   jax.ShapeDtypeStruct((N, D), jnp.float32)],
    )(degp, x, w_in, b_in, w_conv)


def _tc_gate(degp, h, p, b_conv, w_gate, b_gate, w_conv_next):
    return pl.pallas_call(
        _tc_gate_body,
        grid=(N // R,),
        in_specs=[
            _spec_deg(), _spec_rows(),
            pl.BlockSpec((NC, R, D), lambda i: (0, i, 0)),
            _spec_full((1, D)), _spec_full((2 * D, 4)), _spec_full((1, 4)),
            _spec_full((D, D)),
        ],
        out_specs=[_spec_rows(), _spec_rows()],
        out_shape=[jax.ShapeDtypeStruct((N, D), jnp.float32),
                   jax.ShapeDtypeStruct((N, D), jnp.float32)],
    )(degp, h, p, b_conv, w_gate, b_gate, w_conv_next)


def _tc_out(degp, h, p, b_conv, w_gate, b_gate, w_cls, b_cls):
    return pl.pallas_call(
        _tc_out_body,
        grid=(N // R,),
        in_specs=[
            _spec_deg(), _spec_rows(),
            pl.BlockSpec((NC, R, D), lambda i: (0, i, 0)),
            _spec_full((1, D)), _spec_full((2 * D, 4)), _spec_full((1, 4)),
            _spec_full((D, D_OUT)), _spec_full((1, D_OUT)),
        ],
        out_specs=pl.BlockSpec((R, D_OUT), lambda i: (i, 0)),
        out_shape=jax.ShapeDtypeStruct((N, D_OUT), jnp.float32),
    )(degp, h, p, b_conv, w_gate, b_gate, w_cls, b_cls)


def kernel(inputs, graph, W_in, b_in, W_conv0, b_conv0, W_conv1, b_conv1,
           W_gate, b_gate, W_cls, b_cls):
    padw = CH * CK - E // NW
    srcw = jnp.pad(graph[0].reshape(NW, E // NW),
                   ((0, 0), (0, padw))).reshape(NW, CH, CK)
    dummy = jnp.arange(padw, dtype=jnp.int32) % (NP - N) + N
    dstw = jnp.concatenate(
        [graph[1].reshape(NW, E // NW),
         jnp.broadcast_to(dummy, (NW, padw))], axis=1).reshape(NW, CH, CK)
    ones_h = jnp.ones((CK,), jnp.float32)
    zeros_deg = jnp.zeros((RPS,), jnp.float32)
    zeros_row = jnp.zeros((ZR, D), jnp.float32)

    degp = _sc_degrees(srcw, dstw, ones_h, zeros_deg).reshape(NC, 2, NP, 1)

    h, x0 = _tc_in(degp, inputs, W_in, b_in.reshape(1, D), W_conv0)
    p0 = _sc_scatter(x0, graph[0], graph[1], zeros_row)
    h1, x1 = _tc_gate(degp, h, p0, b_conv0.reshape(1, D), W_gate,
                      b_gate.reshape(1, 4), W_conv1)
    p1 = _sc_scatter(x1, graph[0], graph[1], zeros_row)
    out = _tc_out(degp, h1, p1, b_conv1.reshape(1, D), W_gate,
                  b_gate.reshape(1, 4), W_cls, b_cls.reshape(1, D_OUT))
    return out


# speedup vs baseline: 2.3301x; 1.1640x over previous
import functools

import jax
import jax.numpy as jnp
from jax import lax
from jax.experimental import pallas as pl
from jax.experimental.pallas import tpu as pltpu
from jax.experimental.pallas import tpu_sc as plsc

N = 10000
NP = 10240
E = 320000
D = 128
D_OUT = 64

NC = 2
NS = 16
NW = NC * NS
CK = 128
CH = 80
EPW = E // NW
K = 80
EP = NW * CH * CK
DUMP = NP - 8
NB = 2
RPS = NP // NS
ZR = 128

R = 1000


def _mesh():
    return plsc.VectorSubcoreMesh(core_axis_name="c", subcore_axis_name="s")


def _sc_degrees(srcw, dstw, ones_h, zeros_h):
    @functools.partial(
        pl.kernel,
        mesh=_mesh(),
        out_type=jax.ShapeDtypeStruct((NC, 2, NP), jnp.float32),
        scratch_types=[
            pltpu.VMEM((CH, CK), jnp.int32),
            pltpu.VMEM((CH, CK), jnp.int32),
            pltpu.VMEM((CK,), jnp.float32),
            pltpu.VMEM_SHARED((NP,), jnp.float32),
            pltpu.VMEM_SHARED((NP,), jnp.float32),
            pltpu.SemaphoreType.DMA,
        ],
    )
    def deg_kernel(srcw_hbm, dstw_hbm, ones_hbm, zeros_hbm, out_hbm,
                   src_v, dst_v, ones_v, acc_o, acc_i, sem):
        c = lax.axis_index("c")
        s = lax.axis_index("s")
        wid = s * NC + c
        pltpu.sync_copy(zeros_hbm, acc_o.at[pl.ds(s * RPS, RPS)])
        pltpu.sync_copy(zeros_hbm, acc_i.at[pl.ds(s * RPS, RPS)])
        pltpu.sync_copy(srcw_hbm.at[wid], src_v)
        pltpu.sync_copy(dstw_hbm.at[wid], dst_v)
        pltpu.sync_copy(ones_hbm, ones_v)
        plsc.subcore_barrier()

        def step(i, carry):
            pltpu.async_copy(ones_v, acc_o.at[src_v.at[i]], sem, add=True)
            pltpu.async_copy(ones_v, acc_i.at[dst_v.at[i]], sem, add=True)
            return carry

        lax.fori_loop(0, CH, step, 0)

        def drain(i, carry):
            pltpu.make_async_copy(ones_hbm, ones_v, sem).wait()
            return carry

        lax.fori_loop(0, 2 * CH, drain, 0)
        plsc.subcore_barrier()
        pltpu.sync_copy(acc_o.at[pl.ds(s * RPS, RPS)],
                        out_hbm.at[c, 0, pl.ds(s * RPS, RPS)])
        pltpu.sync_copy(acc_i.at[pl.ds(s * RPS, RPS)],
                        out_hbm.at[c, 1, pl.ds(s * RPS, RPS)])

    return deg_kernel(srcw, dstw, ones_h, zeros_h)


def _sc_scatter(x, sd_h, zeros_h):
    nch = EPW // K

    @functools.partial(
        pl.kernel,
        mesh=_mesh(),
        out_type=jax.ShapeDtypeStruct((NC, NP, D), jnp.float32),
        scratch_types=[
            pltpu.VMEM((2, K), jnp.int32),
            pltpu.VMEM((2, K), jnp.int32),
            pltpu.VMEM((K, D), jnp.float32),
            pltpu.VMEM((K, D), jnp.float32),
            pltpu.VMEM_SHARED((NP, D), jnp.float32),
            pltpu.SemaphoreType.DMA,
            pltpu.SemaphoreType.DMA,
        ],
    )
    def scat_kernel(x_hbm, sd_hbm, zeros_hbm, out_hbm,
                    ia, ib, bufa, bufb, acc, sema, semb):
        c = lax.axis_index("c")
        s = lax.axis_index("s")
        wid = s * NC + c

        def load_idx(i, iv):
            pltpu.sync_copy(sd_hbm.at[wid, i], iv)

        def wait_rows(buf, sem):
            pltpu.make_async_copy(x_hbm.at[pl.ds(0, K)], buf, sem).wait()

        for z in range(RPS // ZR):
            pltpu.sync_copy(zeros_hbm, acc.at[pl.ds(s * RPS + z * ZR, ZR)])
        load_idx(0, ia)
        pltpu.async_copy(x_hbm.at[ia.at[0]], bufa, sema)
        load_idx(1, ib)
        pltpu.async_copy(x_hbm.at[ib.at[0]], bufb, semb)
        plsc.subcore_barrier()

        def pair(g, carry):
            wait_rows(bufa, sema)
            pltpu.sync_copy(bufa, acc.at[ia.at[1]], add=True)
            load_idx(2 * g + 2, ia)
            pltpu.async_copy(x_hbm.at[ia.at[0]], bufa, sema)
            wait_rows(bufb, semb)
            pltpu.sync_copy(bufb, acc.at[ib.at[1]], add=True)

            @pl.when(2 * g + 3 < nch)
            def _():
                load_idx(2 * g + 3, ib)
                pltpu.async_copy(x_hbm.at[ib.at[0]], bufb, semb)

            return carry

        lax.fori_loop(0, (nch - 1) // 2, pair, 0)
        wait_rows(bufa, sema)
        pltpu.sync_copy(bufa, acc.at[ia.at[1]], add=True)
        plsc.subcore_barrier()
        pltpu.sync_copy(acc.at[pl.ds(s * RPS, RPS)],
                        out_hbm.at[c, pl.ds(s * RPS, RPS)])

    return scat_kernel(x, sd_h, zeros_h)


def _scales(deg_ref):
    s_out = lax.rsqrt(jnp.maximum(deg_ref[0, 0] + deg_ref[1, 0], 1.0))
    s_in = lax.rsqrt(jnp.maximum(deg_ref[0, 1] + deg_ref[1, 1], 1.0))
    return s_out, s_in


def _dot(a, b):
    return jnp.dot(a, b, preferred_element_type=jnp.float32)


def _tc_in_body(deg_ref, x_ref, win_ref, bin_ref, wc_ref, h_ref, x0_ref):
    s_out, _ = _scales(deg_ref)
    h = jnp.maximum(_dot(x_ref[...], win_ref[...]) + bin_ref[...], 0.0)
    h_ref[...] = h
    x0_ref[...] = _dot(h * s_out, wc_ref[...])


def _tc_gate_body(deg_ref, h_ref, p_ref, bc_ref, wg_ref, bg_ref, wc_ref,
                  h1_ref, x1_ref):
    s_out, s_in = _scales(deg_ref)
    cell = (p_ref[0] + p_ref[1]) * s_in + bc_ref[...]
    h = h_ref[...]
    wg = wg_ref[...]
    cc = jax.nn.sigmoid(_dot(h, wg[0:D]) + _dot(cell, wg[D:2 * D])
                        + bg_ref[...])
    h1_ref[...] = h * cc[:, 0:1] + cell * cc[:, 1:2]
    x1_ref[...] = _dot(cell * cc[:, 3:4] * s_out, wc_ref[...])


def _tc_out_body(deg_ref, h_ref, p_ref, bc_ref, wg_ref, bg_ref, wcls_ref,
                 bcls_ref, out_ref):
    _, s_in = _scales(deg_ref)
    cell = (p_ref[0] + p_ref[1]) * s_in + bc_ref[...]
    h = h_ref[...]
    wg = wg_ref[...]
    cc = jax.nn.sigmoid(_dot(h, wg[0:D]) + _dot(cell, wg[D:2 * D])
                        + bg_ref[...])
    h2 = h * cc[:, 0:1] + cell * cc[:, 1:2]
    out_ref[...] = _dot(h2, wcls_ref[...]) + bcls_ref[...]


def _spec_deg():
    return pl.BlockSpec((NC, 2, R, 1), lambda i: (0, 0, i, 0))


def _spec_rows():
    return pl.BlockSpec((R, D), lambda i: (i, 0))


def _spec_full(shape):
    nd = len(shape)
    return pl.BlockSpec(shape, lambda i: (0,) * nd)


def _tc_in(degp, x, w_in, b_in, w_conv):
    return pl.pallas_call(
        _tc_in_body,
        grid=(N // R,),
        in_specs=[
            _spec_deg(), _spec_rows(),
            _spec_full((D, D)), _spec_full((1, D)), _spec_full((D, D)),
        ],
        out_specs=[_spec_rows(), _spec_rows()],
        out_shape=[jax.ShapeDtypeStruct((N, D), jnp.float32),
                   jax.ShapeDtypeStruct((N, D), jnp.float32)],
    )(degp, x, w_in, b_in, w_conv)


def _tc_gate(degp, h, p, b_conv, w_gate, b_gate, w_conv_next):
    return pl.pallas_call(
        _tc_gate_body,
        grid=(N // R,),
        in_specs=[
            _spec_deg(), _spec_rows(),
            pl.BlockSpec((NC, R, D), lambda i: (0, i, 0)),
            _spec_full((1, D)), _spec_full((2 * D, 4)), _spec_full((1, 4)),
            _spec_full((D, D)),
        ],
        out_specs=[_spec_rows(), _spec_rows()],
        out_shape=[jax.ShapeDtypeStruct((N, D), jnp.float32),
                   jax.ShapeDtypeStruct((N, D), jnp.float32)],
    )(degp, h, p, b_conv, w_gate, b_gate, w_conv_next)


def _tc_out(degp, h, p, b_conv, w_gate, b_gate, w_cls, b_cls):
    return pl.pallas_call(
        _tc_out_body,
        grid=(N // R,),
        in_specs=[
            _spec_deg(), _spec_rows(),
            pl.BlockSpec((NC, R, D), lambda i: (0, i, 0)),
            _spec_full((1, D)), _spec_full((2 * D, 4)), _spec_full((1, 4)),
            _spec_full((D, D_OUT)), _spec_full((1, D_OUT)),
        ],
        out_specs=pl.BlockSpec((R, D_OUT), lambda i: (i, 0)),
        out_shape=jax.ShapeDtypeStruct((N, D_OUT), jnp.float32),
    )(degp, h, p, b_conv, w_gate, b_gate, w_cls, b_cls)


def kernel(inputs, graph, W_in, b_in, W_conv0, b_conv0, W_conv1, b_conv1,
           W_gate, b_gate, W_cls, b_cls):
    padw = CH * CK - E // NW
    srcw = jnp.pad(graph[0].reshape(NW, E // NW),
                   ((0, 0), (0, padw))).reshape(NW, CH, CK)
    dummy = jnp.arange(padw, dtype=jnp.int32) % (NP - N) + N
    dstw = jnp.concatenate(
        [graph[1].reshape(NW, E // NW),
         jnp.broadcast_to(dummy, (NW, padw))], axis=1).reshape(NW, CH, CK)
    ones_h = jnp.ones((CK,), jnp.float32)
    zeros_deg = jnp.zeros((RPS,), jnp.float32)
    zeros_row = jnp.zeros((ZR, D), jnp.float32)
    sd_h = jnp.stack([graph[0].reshape(NW, EPW // K, K),
                      graph[1].reshape(NW, EPW // K, K)], axis=2)

    degp = _sc_degrees(srcw, dstw, ones_h, zeros_deg).reshape(NC, 2, NP, 1)

    h, x0 = _tc_in(degp, inputs, W_in, b_in.reshape(1, D), W_conv0)
    p0 = _sc_scatter(x0, sd_h, zeros_row)
    h1, x1 = _tc_gate(degp, h, p0, b_conv0.reshape(1, D), W_gate,
                      b_gate.reshape(1, 4), W_conv1)
    p1 = _sc_scatter(x1, sd_h, zeros_row)
    out = _tc_out(degp, h1, p1, b_conv1.reshape(1, D), W_gate,
                  b_gate.reshape(1, 4), W_cls, b_cls.reshape(1, D_OUT))
    return out


# speedup vs baseline: 2.7123x; 1.1640x over previous
import functools

import jax
import jax.numpy as jnp
from jax import lax
from jax.experimental import pallas as pl
from jax.experimental.pallas import tpu as pltpu
from jax.experimental.pallas import tpu_sc as plsc

N = 10000
NP = 10240
E = 320000
D = 128
D_OUT = 64

NC = 2
NS = 16
NW = NC * NS
CK = 128
CH = 80
EPW = E // NW
K = 80
EP = NW * CH * CK
DUMP = NP - 8
NB = 2
RPS = NP // NS
ZR = 128

R = 1000


def _mesh():
    return plsc.VectorSubcoreMesh(core_axis_name="c", subcore_axis_name="s")


def _sc_degrees(srcw, dstw, ones_h, zeros_h):
    @functools.partial(
        pl.kernel,
        mesh=_mesh(),
        out_type=jax.ShapeDtypeStruct((NC, 2, NP), jnp.float32),
        scratch_types=[
            pltpu.VMEM((CH, CK), jnp.int32),
            pltpu.VMEM((CH, CK), jnp.int32),
            pltpu.VMEM((CK,), jnp.float32),
            pltpu.VMEM_SHARED((NP,), jnp.float32),
            pltpu.VMEM_SHARED((NP,), jnp.float32),
            pltpu.SemaphoreType.DMA,
        ],
    )
    def deg_kernel(srcw_hbm, dstw_hbm, ones_hbm, zeros_hbm, out_hbm,
                   src_v, dst_v, ones_v, acc_o, acc_i, sem):
        c = lax.axis_index("c")
        s = lax.axis_index("s")
        wid = s * NC + c
        pltpu.sync_copy(zeros_hbm, acc_o.at[pl.ds(s * RPS, RPS)])
        pltpu.sync_copy(zeros_hbm, acc_i.at[pl.ds(s * RPS, RPS)])
        pltpu.sync_copy(srcw_hbm.at[wid], src_v)
        pltpu.sync_copy(dstw_hbm.at[wid], dst_v)
        pltpu.sync_copy(ones_hbm, ones_v)
        plsc.subcore_barrier()

        def step(i, carry):
            pltpu.async_copy(ones_v, acc_o.at[src_v.at[i]], sem, add=True)
            pltpu.async_copy(ones_v, acc_i.at[dst_v.at[i]], sem, add=True)
            return carry

        lax.fori_loop(0, CH, step, 0)

        def drain(i, carry):
            pltpu.make_async_copy(ones_hbm, ones_v, sem).wait()
            return carry

        lax.fori_loop(0, 2 * CH, drain, 0)
        plsc.subcore_barrier()
        pltpu.sync_copy(acc_o.at[pl.ds(s * RPS, RPS)],
                        out_hbm.at[c, 0, pl.ds(s * RPS, RPS)])
        pltpu.sync_copy(acc_i.at[pl.ds(s * RPS, RPS)],
                        out_hbm.at[c, 1, pl.ds(s * RPS, RPS)])

    return deg_kernel(srcw, dstw, ones_h, zeros_h)


def _sc_scatter(x, sd_h, zeros_h):
    nch = EPW // K

    @functools.partial(
        pl.kernel,
        mesh=_mesh(),
        out_type=jax.ShapeDtypeStruct((NC, NP, D), jnp.float32),
        scratch_types=[
            pltpu.VMEM((2, K), jnp.int32),
            pltpu.VMEM((2, K), jnp.int32),
            pltpu.VMEM((2, K), jnp.int32),
            pltpu.VMEM((2, K), jnp.int32),
            pltpu.VMEM((K, D), jnp.float32),
            pltpu.VMEM((K, D), jnp.float32),
            pltpu.VMEM_SHARED((NP, D), jnp.float32),
        ] + [pltpu.SemaphoreType.DMA for _ in range(6)],
    )
    def scat_kernel(x_hbm, sd_hbm, zeros_hbm, out_hbm,
                    ia0, ia1, ib0, ib1, bufa, bufb, acc,
                    sia0, sia1, sib0, sib1, sga, sgb):
        iv = ((ia0, ia1), (ib0, ib1))
        isem = ((sia0, sia1), (sib0, sib1))
        gsem = (sga, sgb)
        bufs = (bufa, bufb)
        c = lax.axis_index("c")
        s = lax.axis_index("s")
        wid = s * NC + c

        def wait_idx(b, p):
            pltpu.make_async_copy(sd_hbm.at[0, 0], iv[b][p], isem[b][p]).wait()

        def wait_rows(b):
            pltpu.make_async_copy(x_hbm.at[pl.ds(0, K)], bufs[b],
                                  gsem[b]).wait()

        for z in range(RPS // ZR):
            pltpu.sync_copy(zeros_hbm, acc.at[pl.ds(s * RPS + z * ZR, ZR)])
        pltpu.sync_copy(sd_hbm.at[wid, 0], iv[0][0])
        pltpu.sync_copy(sd_hbm.at[wid, 1], iv[1][0])
        pltpu.async_copy(sd_hbm.at[wid, 2], iv[0][1], isem[0][1])
        pltpu.async_copy(sd_hbm.at[wid, 3], iv[1][1], isem[1][1])
        pltpu.async_copy(x_hbm.at[iv[0][0].at[0]], bufa, sga)
        pltpu.async_copy(x_hbm.at[iv[1][0].at[0]], bufb, sgb)
        plsc.subcore_barrier()

        def quad(g, carry):
            for t in range(4):
                b = t % 2
                p = (t // 2) % 2
                i = 4 * g + t
                wait_rows(b)
                pltpu.sync_copy(bufs[b], acc.at[iv[b][p].at[1]], add=True)

                @pl.when(i + 4 < nch)
                def _():
                    pltpu.async_copy(sd_hbm.at[wid, i + 4], iv[b][p],
                                     isem[b][p])

                @pl.when(i + 2 < nch)
                def _():
                    wait_idx(b, 1 - p)
                    pltpu.async_copy(x_hbm.at[iv[b][1 - p].at[0]], bufs[b],
                                     gsem[b])

            return carry

        lax.fori_loop(0, nch // 4, quad, 0)
        wait_rows(0)
        pltpu.sync_copy(bufa, acc.at[iv[0][0].at[1]], add=True)
        plsc.subcore_barrier()
        pltpu.sync_copy(acc.at[pl.ds(s * RPS, RPS)],
                        out_hbm.at[c, pl.ds(s * RPS, RPS)])

    return scat_kernel(x, sd_h, zeros_h)


def _scales(deg_ref):
    s_out = lax.rsqrt(jnp.maximum(deg_ref[0, 0] + deg_ref[1, 0], 1.0))
    s_in = lax.rsqrt(jnp.maximum(deg_ref[0, 1] + deg_ref[1, 1], 1.0))
    return s_out, s_in


def _dot(a, b):
    return jnp.dot(a, b, preferred_element_type=jnp.float32)


def _tc_in_body(deg_ref, x_ref, win_ref, bin_ref, wc_ref, h_ref, x0_ref):
    s_out, _ = _scales(deg_ref)
    h = jnp.maximum(_dot(x_ref[...], win_ref[...]) + bin_ref[...], 0.0)
    h_ref[...] = h
    x0_ref[...] = _dot(h * s_out, wc_ref[...])


def _tc_gate_body(deg_ref, h_ref, p_ref, bc_ref, wg_ref, bg_ref, wc_ref,
                  h1_ref, x1_ref):
    s_out, s_in = _scales(deg_ref)
    cell = (p_ref[0] + p_ref[1]) * s_in + bc_ref[...]
    h = h_ref[...]
    wg = wg_ref[...]
    cc = jax.nn.sigmoid(_dot(h, wg[0:D]) + _dot(cell, wg[D:2 * D])
                        + bg_ref[...])
    h1_ref[...] = h * cc[:, 0:1] + cell * cc[:, 1:2]
    x1_ref[...] = _dot(cell * cc[:, 3:4] * s_out, wc_ref[...])


def _tc_out_body(deg_ref, h_ref, p_ref, bc_ref, wg_ref, bg_ref, wcls_ref,
                 bcls_ref, out_ref):
    _, s_in = _scales(deg_ref)
    cell = (p_ref[0] + p_ref[1]) * s_in + bc_ref[...]
    h = h_ref[...]
    wg = wg_ref[...]
    cc = jax.nn.sigmoid(_dot(h, wg[0:D]) + _dot(cell, wg[D:2 * D])
                        + bg_ref[...])
    h2 = h * cc[:, 0:1] + cell * cc[:, 1:2]
    out_ref[...] = _dot(h2, wcls_ref[...]) + bcls_ref[...]


def _spec_deg():
    return pl.BlockSpec((NC, 2, R, 1), lambda i: (0, 0, i, 0))


def _spec_rows():
    return pl.BlockSpec((R, D), lambda i: (i, 0))


def _spec_full(shape):
    nd = len(shape)
    return pl.BlockSpec(shape, lambda i: (0,) * nd)


def _tc_in(degp, x, w_in, b_in, w_conv):
    return pl.pallas_call(
        _tc_in_body,
        grid=(N // R,),
        in_specs=[
            _spec_deg(), _spec_rows(),
            _spec_full((D, D)), _spec_full((1, D)), _spec_full((D, D)),
        ],
        out_specs=[_spec_rows(), _spec_rows()],
        out_shape=[jax.ShapeDtypeStruct((N, D), jnp.float32),
                   jax.ShapeDtypeStruct((N, D), jnp.float32)],
    )(degp, x, w_in, b_in, w_conv)


def _tc_gate(degp, h, p, b_conv, w_gate, b_gate, w_conv_next):
    return pl.pallas_call(
        _tc_gate_body,
        grid=(N // R,),
        in_specs=[
            _spec_deg(), _spec_rows(),
            pl.BlockSpec((NC, R, D), lambda i: (0, i, 0)),
            _spec_full((1, D)), _spec_full((2 * D, 4)), _spec_full((1, 4)),
            _spec_full((D, D)),
        ],
        out_specs=[_spec_rows(), _spec_rows()],
        out_shape=[jax.ShapeDtypeStruct((N, D), jnp.float32),
                   jax.ShapeDtypeStruct((N, D), jnp.float32)],
    )(degp, h, p, b_conv, w_gate, b_gate, w_conv_next)


def _tc_out(degp, h, p, b_conv, w_gate, b_gate, w_cls, b_cls):
    return pl.pallas_call(
        _tc_out_body,
        grid=(N // R,),
        in_specs=[
            _spec_deg(), _spec_rows(),
            pl.BlockSpec((NC, R, D), lambda i: (0, i, 0)),
            _spec_full((1, D)), _spec_full((2 * D, 4)), _spec_full((1, 4)),
            _spec_full((D, D_OUT)), _spec_full((1, D_OUT)),
        ],
        out_specs=pl.BlockSpec((R, D_OUT), lambda i: (i, 0)),
        out_shape=jax.ShapeDtypeStruct((N, D_OUT), jnp.float32),
    )(degp, h, p, b_conv, w_gate, b_gate, w_cls, b_cls)


def kernel(inputs, graph, W_in, b_in, W_conv0, b_conv0, W_conv1, b_conv1,
           W_gate, b_gate, W_cls, b_cls):
    padw = CH * CK - E // NW
    srcw = jnp.pad(graph[0].reshape(NW, E // NW),
                   ((0, 0), (0, padw))).reshape(NW, CH, CK)
    dummy = jnp.arange(padw, dtype=jnp.int32) % (NP - N) + N
    dstw = jnp.concatenate(
        [graph[1].reshape(NW, E // NW),
         jnp.broadcast_to(dummy, (NW, padw))], axis=1).reshape(NW, CH, CK)
    ones_h = jnp.ones((CK,), jnp.float32)
    zeros_deg = jnp.zeros((RPS,), jnp.float32)
    zeros_row = jnp.zeros((ZR, D), jnp.float32)
    sd_h = jnp.stack([graph[0].reshape(NW, EPW // K, K),
                      graph[1].reshape(NW, EPW // K, K)], axis=2)

    degp = _sc_degrees(srcw, dstw, ones_h, zeros_deg).reshape(NC, 2, NP, 1)

    h, x0 = _tc_in(degp, inputs, W_in, b_in.reshape(1, D), W_conv0)
    p0 = _sc_scatter(x0, sd_h, zeros_row)
    h1, x1 = _tc_gate(degp, h, p0, b_conv0.reshape(1, D), W_gate,
                      b_gate.reshape(1, 4), W_conv1)
    p1 = _sc_scatter(x1, sd_h, zeros_row)
    out = _tc_out(degp, h1, p1, b_conv1.reshape(1, D), W_gate,
                  b_gate.reshape(1, 4), W_cls, b_cls.reshape(1, D_OUT))
    return out


# speedup vs baseline: 2.7174x; 1.0019x over previous
import functools

import jax
import jax.numpy as jnp
from jax import lax
from jax.experimental import pallas as pl
from jax.experimental.pallas import tpu as pltpu
from jax.experimental.pallas import tpu_sc as plsc

N = 10000
NP = 10240
E = 320000
D = 128
D_OUT = 64

NC = 2
NS = 16
NW = NC * NS
CK = 128
CH = 80
EPW = E // NW
K = 80
RPS = NP // NS
ZR = 128

R = 1000


def _mesh():
    return plsc.VectorSubcoreMesh(core_axis_name="c", subcore_axis_name="s")


def _sc_degrees(srcw, dstw, ones_h, zeros_h):
    @functools.partial(
        pl.kernel,
        mesh=_mesh(),
        out_type=jax.ShapeDtypeStruct((NC, 2, NP), jnp.float32),
        scratch_types=[
            pltpu.VMEM((CH, CK), jnp.int32),
            pltpu.VMEM((CH, CK), jnp.int32),
            pltpu.VMEM((CK,), jnp.float32),
            pltpu.VMEM_SHARED((NP,), jnp.float32),
            pltpu.VMEM_SHARED((NP,), jnp.float32),
            pltpu.SemaphoreType.DMA,
        ],
    )
    def deg_kernel(srcw_hbm, dstw_hbm, ones_hbm, zeros_hbm, out_hbm,
                   src_v, dst_v, ones_v, acc_o, acc_i, sem):
        c = lax.axis_index("c")
        s = lax.axis_index("s")
        wid = s * NC + c
        pltpu.sync_copy(zeros_hbm, acc_o.at[pl.ds(s * RPS, RPS)])
        pltpu.sync_copy(zeros_hbm, acc_i.at[pl.ds(s * RPS, RPS)])
        pltpu.sync_copy(srcw_hbm.at[wid], src_v)
        pltpu.sync_copy(dstw_hbm.at[wid], dst_v)
        pltpu.sync_copy(ones_hbm, ones_v)
        plsc.subcore_barrier()

        def step(i, carry):
            pltpu.async_copy(ones_v, acc_o.at[src_v.at[i]], sem, add=True)
            pltpu.async_copy(ones_v, acc_i.at[dst_v.at[i]], sem, add=True)
            return carry

        lax.fori_loop(0, CH, step, 0)

        def drain(i, carry):
            pltpu.make_async_copy(ones_hbm, ones_v, sem).wait()
            return carry

        lax.fori_loop(0, 2 * CH, drain, 0)
        plsc.subcore_barrier()
        pltpu.sync_copy(acc_o.at[pl.ds(s * RPS, RPS)],
                        out_hbm.at[c, 0, pl.ds(s * RPS, RPS)])
        pltpu.sync_copy(acc_i.at[pl.ds(s * RPS, RPS)],
                        out_hbm.at[c, 1, pl.ds(s * RPS, RPS)])

    return deg_kernel(srcw, dstw, ones_h, zeros_h)


def _sc_scatter(x, sd_h, zeros_h):
    nch = EPW // K

    @functools.partial(
        pl.kernel,
        mesh=_mesh(),
        out_type=jax.ShapeDtypeStruct((NC, NP, D), jnp.float32),
        scratch_types=[
            pltpu.VMEM((2, K), jnp.int32),
            pltpu.VMEM((2, K), jnp.int32),
            pltpu.VMEM((2, K), jnp.int32),
            pltpu.VMEM((2, K), jnp.int32),
            pltpu.VMEM((K, D), jnp.float32),
            pltpu.VMEM((K, D), jnp.float32),
            pltpu.VMEM_SHARED((NP, D), jnp.float32),
        ] + [pltpu.SemaphoreType.DMA for _ in range(6)],
    )
    def scat_kernel(x_hbm, sd_hbm, zeros_hbm, out_hbm,
                    ia0, ia1, ib0, ib1, bufa, bufb, acc,
                    sia0, sia1, sib0, sib1, sga, sgb):
        iv = ((ia0, ia1), (ib0, ib1))
        isem = ((sia0, sia1), (sib0, sib1))
        gsem = (sga, sgb)
        bufs = (bufa, bufb)
        c = lax.axis_index("c")
        s = lax.axis_index("s")
        wid = s * NC + c

        def wait_idx(b, p):
            pltpu.make_async_copy(sd_hbm.at[0, 0], iv[b][p], isem[b][p]).wait()

        def wait_rows(b):
            pltpu.make_async_copy(x_hbm.at[pl.ds(0, K)], bufs[b],
                                  gsem[b]).wait()

        for z in range(RPS // ZR):
            pltpu.sync_copy(zeros_hbm, acc.at[pl.ds(s * RPS + z * ZR, ZR)])
        pltpu.sync_copy(sd_hbm.at[wid, 0], iv[0][0])
        pltpu.sync_copy(sd_hbm.at[wid, 1], iv[1][0])
        pltpu.async_copy(sd_hbm.at[wid, 2], iv[0][1], isem[0][1])
        pltpu.async_copy(sd_hbm.at[wid, 3], iv[1][1], isem[1][1])
        pltpu.async_copy(x_hbm.at[iv[0][0].at[0]], bufa, sga)
        pltpu.async_copy(x_hbm.at[iv[1][0].at[0]], bufb, sgb)
        plsc.subcore_barrier()

        def quad(g, carry):
            for t in range(4):
                b = t % 2
                p = (t // 2) % 2
                i = 4 * g + t
                wait_rows(b)
                pltpu.sync_copy(bufs[b], acc.at[iv[b][p].at[1]], add=True)

                @pl.when(i + 4 < nch)
                def _():
                    pltpu.async_copy(sd_hbm.at[wid, i + 4], iv[b][p],
                                     isem[b][p])

                @pl.when(i + 2 < nch)
                def _():
                    wait_idx(b, 1 - p)
                    pltpu.async_copy(x_hbm.at[iv[b][1 - p].at[0]], bufs[b],
                                     gsem[b])

            return carry

        lax.fori_loop(0, nch // 4, quad, 0)
        wait_rows(0)
        pltpu.sync_copy(bufa, acc.at[iv[0][0].at[1]], add=True)
        plsc.subcore_barrier()
        pltpu.sync_copy(acc.at[pl.ds(s * RPS, RPS)],
                        out_hbm.at[c, pl.ds(s * RPS, RPS)])

    return scat_kernel(x, sd_h, zeros_h)


def _scales(deg_ref):
    s_out = lax.rsqrt(jnp.maximum(deg_ref[0, 0] + deg_ref[1, 0], 1.0))
    s_in = lax.rsqrt(jnp.maximum(deg_ref[0, 1] + deg_ref[1, 1], 1.0))
    return s_out, s_in


def _dot(a, b):
    return jnp.dot(a, b, preferred_element_type=jnp.float32)


def _tc_in_body(deg_ref, x_ref, win_ref, bin_ref, wc_ref, h_ref, x0_ref):
    s_out, _ = _scales(deg_ref)
    h = jnp.maximum(_dot(x_ref[...], win_ref[...]) + bin_ref[...], 0.0)
    h_ref[...] = h
    x0_ref[...] = _dot(h * s_out, wc_ref[...])


def _tc_gate_body(deg_ref, h_ref, p_ref, bc_ref, wg_ref, bg_ref, wc_ref,
                  h1_ref, x1_ref):
    s_out, s_in = _scales(deg_ref)
    cell = (p_ref[0] + p_ref[1]) * s_in + bc_ref[...]
    h = h_ref[...]
    wg = wg_ref[...]
    cc = jax.nn.sigmoid(_dot(h, wg[0:D]) + _dot(cell, wg[D:2 * D])
                        + bg_ref[...])
    h1_ref[...] = h * cc[:, 0:1] + cell * cc[:, 1:2]
    x1_ref[...] = _dot(cell * cc[:, 3:4] * s_out, wc_ref[...])


def _tc_out_body(deg_ref, h_ref, p_ref, bc_ref, wg_ref, bg_ref, wcls_ref,
                 bcls_ref, out_ref):
    _, s_in = _scales(deg_ref)
    cell = (p_ref[0] + p_ref[1]) * s_in + bc_ref[...]
    h = h_ref[...]
    wg = wg_ref[...]
    cc = jax.nn.sigmoid(_dot(h, wg[0:D]) + _dot(cell, wg[D:2 * D])
                        + bg_ref[...])
    h2 = h * cc[:, 0:1] + cell * cc[:, 1:2]
    out_ref[...] = _dot(h2, wcls_ref[...]) + bcls_ref[...]


def _spec_deg():
    return pl.BlockSpec((NC, 2, R, 1), lambda i: (0, 0, i, 0))


def _spec_rows():
    return pl.BlockSpec((R, D), lambda i: (i, 0))


def _spec_full(shape):
    nd = len(shape)
    return pl.BlockSpec(shape, lambda i: (0,) * nd)


def _tc_in(degp, x, w_in, b_in, w_conv):
    return pl.pallas_call(
        _tc_in_body,
        grid=(N // R,),
        in_specs=[
            _spec_deg(), _spec_rows(),
            _spec_full((D, D)), _spec_full((1, D)), _spec_full((D, D)),
        ],
        out_specs=[_spec_rows(), _spec_rows()],
        out_shape=[jax.ShapeDtypeStruct((N, D), jnp.float32),
                   jax.ShapeDtypeStruct((N, D), jnp.float32)],
    )(degp, x, w_in, b_in, w_conv)


def _tc_gate(degp, h, p, b_conv, w_gate, b_gate, w_conv_next):
    return pl.pallas_call(
        _tc_gate_body,
        grid=(N // R,),
        in_specs=[
            _spec_deg(), _spec_rows(),
            pl.BlockSpec((NC, R, D), lambda i: (0, i, 0)),
            _spec_full((1, D)), _spec_full((2 * D, 4)), _spec_full((1, 4)),
            _spec_full((D, D)),
        ],
        out_specs=[_spec_rows(), _spec_rows()],
        out_shape=[jax.ShapeDtypeStruct((N, D), jnp.float32),
                   jax.ShapeDtypeStruct((N, D), jnp.float32)],
    )(degp, h, p, b_conv, w_gate, b_gate, w_conv_next)


def _tc_out(degp, h, p, b_conv, w_gate, b_gate, w_cls, b_cls):
    return pl.pallas_call(
        _tc_out_body,
        grid=(N // R,),
        in_specs=[
            _spec_deg(), _spec_rows(),
            pl.BlockSpec((NC, R, D), lambda i: (0, i, 0)),
            _spec_full((1, D)), _spec_full((2 * D, 4)), _spec_full((1, 4)),
            _spec_full((D, D_OUT)), _spec_full((1, D_OUT)),
        ],
        out_specs=pl.BlockSpec((R, D_OUT), lambda i: (i, 0)),
        out_shape=jax.ShapeDtypeStruct((N, D_OUT), jnp.float32),
    )(degp, h, p, b_conv, w_gate, b_gate, w_cls, b_cls)


def kernel(inputs, graph, W_in, b_in, W_conv0, b_conv0, W_conv1, b_conv1,
           W_gate, b_gate, W_cls, b_cls):
    padw = CH * CK - E // NW
    srcw = jnp.pad(graph[0].reshape(NW, E // NW),
                   ((0, 0), (0, padw))).reshape(NW, CH, CK)
    dummy = jnp.arange(padw, dtype=jnp.int32) % (NP - N) + N
    dstw = jnp.concatenate(
        [graph[1].reshape(NW, E // NW),
         jnp.broadcast_to(dummy, (NW, padw))], axis=1).reshape(NW, CH, CK)
    ones_h = jnp.ones((CK,), jnp.float32)
    zeros_deg = jnp.zeros((RPS,), jnp.float32)
    zeros_row = jnp.zeros((ZR, D), jnp.float32)
    sd_h = jnp.stack([graph[0].reshape(NW, EPW // K, K),
                      graph[1].reshape(NW, EPW // K, K)], axis=2)

    degp = _sc_degrees(srcw, dstw, ones_h, zeros_deg).reshape(NC, 2, NP, 1)

    h, x0 = _tc_in(degp, inputs, W_in, b_in.reshape(1, D), W_conv0)
    p0 = _sc_scatter(x0, sd_h, zeros_row)
    h1, x1 = _tc_gate(degp, h, p0, b_conv0.reshape(1, D), W_gate,
                      b_gate.reshape(1, 4), W_conv1)
    p1 = _sc_scatter(x1, sd_h, zeros_row)
    out = _tc_out(degp, h1, p1, b_conv1.reshape(1, D), W_gate,
                  b_gate.reshape(1, 4), W_cls, b_cls.reshape(1, D_OUT))
    return out
